# baseline (speedup 1.0000x reference)
import functools

import jax
import jax.numpy as jnp
from jax import lax
from jax.experimental import pallas as pl
from jax.experimental.pallas import tpu as pltpu
from jax.experimental.pallas import tpu_sc as plsc

D = 64
M = 128

_NC = 2
_NS = 16
_NW = _NC * _NS

_CHUNK = 160
_PBLK = 32768


def _pt_main_body(vt_ref, w_ref, o_ref):
    o_ref[...] = lax.dot_general(
        vt_ref[...], w_ref[...],
        dimension_numbers=(((0,), (1,)), ((), ())),
        preferred_element_type=jnp.float32)


def _project_table(vt, w):
    v = vt.shape[1]
    return pl.pallas_call(
        _pt_main_body,
        grid=(pl.cdiv(v, _PBLK),),
        in_specs=[
            pl.BlockSpec((D, _PBLK), lambda i: (0, i)),
            pl.BlockSpec((M, D), lambda i: (0, 0)),
        ],
        out_specs=pl.BlockSpec((_PBLK, M), lambda i: (i, 0)),
        out_shape=jax.ShapeDtypeStruct((v, M), jnp.float32),
    )(vt, w)


def _gather_body(table_hbm, ids_hbm, out_hbm, idx_v,
                 rows0, rows1, rows2, rows3,
                 sg0, sg1, sg2, sg3, sw0, sw1, sw2, sw3):
    wid = lax.axis_index("s") * _NC + lax.axis_index("c")
    n = ids_hbm.shape[0]
    b_per_w = n // _NW
    n_chunks = b_per_w // _CHUNK
    base = wid * b_per_w
    rows = (rows0, rows1, rows2, rows3)
    sg = (sg0, sg1, sg2, sg3)
    sw = (sw0, sw1, sw2, sw3)
    nbuf = 4

    def idx_at(g):
        return idx_v.at[pl.ds(g * _CHUNK, _CHUNK)]

    def out_at(g):
        return out_hbm.at[pl.ds(base + g * _CHUNK, _CHUNK)]

    pltpu.sync_copy(ids_hbm.at[pl.ds(base, b_per_w)], idx_v)
    for b in range(nbuf):
        pltpu.async_copy(table_hbm.at[idx_at(b)], rows[b], sg[b])

    def ring(p, carry):
        g0 = p * nbuf
        for b in range(nbuf):
            g = g0 + b
            pltpu.make_async_copy(table_hbm.at[idx_at(g)], rows[b],
                                  sg[b]).wait()
            pltpu.async_copy(rows[b], out_at(g), sw[b])

            @pl.when(g + nbuf < n_chunks)
            def _():
                pltpu.make_async_copy(rows[b], out_at(g), sw[b]).wait()
                pltpu.async_copy(table_hbm.at[idx_at(g + nbuf)], rows[b],
                                 sg[b])
        return carry

    lax.fori_loop(0, n_chunks // nbuf, ring, 0)
    for b in range(nbuf):
        g = n_chunks - nbuf + b
        pltpu.make_async_copy(rows[b], out_at(g), sw[b]).wait()


@functools.cache
def _make_gather(n, v):
    mesh = plsc.VectorSubcoreMesh(core_axis_name="c", subcore_axis_name="s")
    b_per_w = n // _NW
    return pl.kernel(
        _gather_body,
        mesh=mesh,
        out_type=jax.ShapeDtypeStruct((n, M), jnp.float32),
        scratch_types=(
            [pltpu.VMEM((b_per_w,), jnp.int32)]
            + [pltpu.VMEM((_CHUNK, M), jnp.float32)] * 4
            + [pltpu.SemaphoreType.DMA] * 8
        ),
        compiler_params=pltpu.CompilerParams(use_tc_tiling_on_sc=False),
    )


def kernel(input_ids, low_rank_embed, projection_w):
    bsz, seq = input_ids.shape
    ids = input_ids.reshape(-1).astype(jnp.int32)
    n = ids.shape[0]
    v = low_rank_embed.shape[0]
    vt = low_rank_embed.T
    ptable = _project_table(vt, projection_w)
    out = _make_gather(n, v)(ptable, ids)
    return out.reshape(bsz, seq, M)
